# baseline (speedup 1.0000x reference)
import functools

import jax
import jax.numpy as jnp
from jax import lax
from jax.experimental import pallas as pl
from jax.experimental.pallas import tpu as pltpu
from jax.experimental.pallas import tpu_sc as plsc

N = 10000
E = 320000
D = 128
G = 64

NC = 2
NS = 16
NW = NC * NS
CH = 128
ET = E // NW
NCHA = ET // CH
TAIL = ET - NCHA * CH
NPAD = 10240
RPT = NPAD // NS
NBUF = 2

_mesh = plsc.VectorSubcoreMesh(core_axis_name="c", subcore_axis_name="s")


@functools.partial(
    pl.kernel,
    out_type=jax.ShapeDtypeStruct((NC, NPAD, D), jnp.float32),
    mesh=_mesh,
    scratch_types=[
        [pltpu.VMEM((CH,), jnp.int32)] * NBUF,
        pltpu.VMEM((CH,), jnp.int32),
        pltpu.VMEM((TAIL,), jnp.int32),
        [pltpu.VMEM((CH, D), jnp.float32)] * NBUF,
        pltpu.VMEM((TAIL, D), jnp.float32),
        pltpu.VMEM_SHARED((NPAD, D), jnp.float32),
        [pltpu.SemaphoreType.DMA] * NBUF,
        pltpu.SemaphoreType.DMA,
    ],
)
def _sc_aggregate(h_hbm, src_hbm, dst_hbm, zeros_hbm, out_hbm,
                  sidx_f, didx_f, tidx, rows, rows_t, acc, sems, tsem):
    c = lax.axis_index("c")
    s = lax.axis_index("s")
    w = c * NS + s
    row0 = s * RPT
    base = w * ET

    def zinit(k, _):
        pltpu.sync_copy(zeros_hbm, acc.at[pl.ds(row0 + k * CH, CH)])
        return 0
    lax.fori_loop(0, RPT // CH, zinit, 0)
    plsc.subcore_barrier()

    def fire(ch, b):
        pltpu.sync_copy(src_hbm.at[pl.ds(base + ch * CH, CH)], sidx_f[b])
        pltpu.async_copy(h_hbm.at[sidx_f[b]], rows[b], sems[b])

    def drain(ch, b):
        pltpu.make_async_copy(h_hbm.at[sidx_f[b]], rows[b], sems[b]).wait()

    def scat(ch, b):
        pltpu.sync_copy(dst_hbm.at[pl.ds(base + ch * CH, CH)], didx_f)
        pltpu.sync_copy(rows[b], acc.at[didx_f], add=True)

    for b in range(NBUF):
        fire(b, b)

    def step(i, _):
        bs = i * NBUF
        for b in range(NBUF):
            ch = bs + b
            drain(ch, b)
            scat(ch, b)
            fire(ch + NBUF, b)
        return 0
    lax.fori_loop(0, NCHA // NBUF - 2, step, 0)
    bs = NCHA - 2 * NBUF
    for b in range(NBUF):
        ch = bs + b
        drain(ch, b)
        scat(ch, b)
        fire(ch + NBUF, b)
    for b in range(NBUF):
        ch = NCHA - NBUF + b
        drain(ch, b)
        scat(ch, b)

    toff = base + NCHA * CH
    pltpu.sync_copy(src_hbm.at[pl.ds(toff, TAIL)], tidx)
    pltpu.async_copy(h_hbm.at[tidx], rows_t, tsem).wait()
    pltpu.sync_copy(dst_hbm.at[pl.ds(toff, TAIL)], tidx)
    pltpu.sync_copy(rows_t, acc.at[tidx], add=True)

    plsc.subcore_barrier()
    pltpu.sync_copy(acc.at[pl.ds(row0, RPT)], out_hbm.at[c, pl.ds(row0, RPT)])


def _mm_body(x_ref, w_ref, o_ref):
    o_ref[...] = jnp.dot(x_ref[...], w_ref[...],
                         preferred_element_type=jnp.float32)


def _tc_matmul(x, w):
    return pl.pallas_call(
        _mm_body,
        out_shape=jax.ShapeDtypeStruct((x.shape[0], w.shape[1]), jnp.float32),
    )(x, w)


def _scale_body(dp_ref, xw_ref, dis_ref, h1p_ref):
    deg = dp_ref[0, 0:N, 0:1] + dp_ref[1, 0:N, 0:1] + 1.0
    dis = lax.rsqrt(deg)
    dis_ref[...] = dis
    h1p_ref[...] = xw_ref[...] * dis


def _tc_scale(deg_partials, xw):
    return pl.pallas_call(
        _scale_body,
        out_shape=[
            jax.ShapeDtypeStruct((N, 1), jnp.float32),
            jax.ShapeDtypeStruct((N, D), jnp.float32),
        ],
    )(deg_partials, xw)


def _layer_body(ap_ref, hp_ref, dis_ref, b_ref, w_ref, o_ref):
    dis = dis_ref[...]
    t = (ap_ref[0, 0:N] + ap_ref[1, 0:N] + hp_ref[...]) * dis + b_ref[...]
    h = jnp.maximum(t, 0.0)
    o_ref[...] = jnp.dot(h, w_ref[...], preferred_element_type=jnp.float32) * dis


def _tc_layer(agg_partials, hp, dis, b2d, w):
    return pl.pallas_call(
        _layer_body,
        out_shape=jax.ShapeDtypeStruct((N, D), jnp.float32),
    )(agg_partials, hp, dis, b2d, w)


def _final_body(ap_ref, hp_ref, dis_ref, b_ref, batch_ref, wl_ref, bl_ref, o_ref):
    h3 = (ap_ref[0, 0:N] + ap_ref[1, 0:N] + hp_ref[...]) * dis_ref[...] + b_ref[...]
    gids = lax.broadcasted_iota(jnp.int32, (N, G), 1)
    onehot = (batch_ref[...] == gids).astype(jnp.float32)
    pooled = lax.dot_general(onehot, h3, (((0,), (0,)), ((), ())),
                             preferred_element_type=jnp.float32)
    cnt = jnp.sum(onehot, axis=0)[:, None]
    g = pooled / jnp.maximum(cnt, 1.0)
    z = jnp.dot(g, wl_ref[...], preferred_element_type=jnp.float32) + bl_ref[...]
    o_ref[...] = 1.0 / (1.0 + jnp.exp(-z))


def _tc_final(agg_partials, hp, dis, b2d, batch2d, wl, bl2d):
    return pl.pallas_call(
        _final_body,
        out_shape=jax.ShapeDtypeStruct((G, 1), jnp.float32),
    )(agg_partials, hp, dis, b2d, batch2d, wl, bl2d)


def kernel(x, edge_index, batch, W1, b1, W2, b2, W3, b3, Wl, bl):
    src1d = edge_index[0]
    dst1d = edge_index[1]
    batch2d = batch.reshape(N, 1)
    zerosD = jnp.zeros((CH, D), jnp.float32)
    onesND = jnp.ones((N, D), jnp.float32)

    deg_partials = _sc_aggregate(onesND, src1d, dst1d, zerosD)
    xw = _tc_matmul(x, W1)
    dis, hp = _tc_scale(deg_partials, xw)

    agg = _sc_aggregate(hp, src1d, dst1d, zerosD)
    hp = _tc_layer(agg, hp, dis, b1.reshape(1, D), W2)
    agg = _sc_aggregate(hp, src1d, dst1d, zerosD)
    hp = _tc_layer(agg, hp, dis, b2.reshape(1, D), W3)
    agg = _sc_aggregate(hp, src1d, dst1d, zerosD)
    return _tc_final(agg, hp, dis, b3.reshape(1, D), batch2d, Wl,
                     bl.reshape(1, 1))
